# baseline (speedup 1.0000x reference)
import functools

import jax
import jax.numpy as jnp
from jax import lax
from jax.experimental import pallas as pl
from jax.experimental.pallas import tpu as pltpu
from jax.experimental.pallas import tpu_sc as plsc

_N_ROWS = 4096
_N_COLS = 20000
_N_SEL = 2000
_NC = 2
_NS = 16
_L = 16
_NW = _NC * _NS
_MASK_VREGS = _N_COLS // _L
_VPW = -(-_MASK_VREGS // _NS)
_MASK_W = _VPW * _L
_IDX_PAD = 2048
_K = 8
_NCHUNK = 8


def _body(xt_hbm, mask_hbm, out_hbm,
          mask_v, idx_v, accv, cnt_tab, win2_v, win_v, buf0, buf1,
          counts_sh, idx_sh, sem0, sem1, sem2, sem3):
    cid = lax.axis_index("c")
    sid = lax.axis_index("s")
    wid = sid * _NC + cid

    dma_off = jnp.minimum(sid * _MASK_W, _N_COLS - _MASK_W)
    pltpu.sync_copy(mask_hbm.at[pl.ds(dma_off, _MASK_W)], mask_v)

    def init_body(j, carry):
        idx_v[pl.ds(j * _L, _L)] = jnp.zeros((_L,), jnp.int32)
        return carry

    lax.fori_loop(0, _IDX_PAD // _L, init_body, jnp.int32(0), unroll=False)

    gstart = sid * _VPW
    trip = jnp.minimum(_MASK_VREGS - gstart, _VPW)
    lbase_w = gstart * _L - dma_off

    def pass_a(t, acc):
        mv = mask_v[pl.ds(lbase_w + t * _L, _L)]
        return acc + (mv != 0.0).astype(jnp.int32)

    acc = lax.fori_loop(0, trip, pass_a, jnp.zeros((_L,), jnp.int32),
                        unroll=False)
    accv[...] = acc
    pltpu.sync_copy(accv, counts_sh.at[sid])
    plsc.subcore_barrier()

    pltpu.sync_copy(counts_sh, cnt_tab)
    acc2 = jnp.zeros((_L,), jnp.int32)
    for r in range(_NS):
        acc2 = acc2 + cnt_tab[r, :] * (r < sid).astype(jnp.int32)
    base = jnp.sum(acc2)

    def pass_b(t, off):
        mv = mask_v[pl.ds(lbase_w + t * _L, _L)]
        m = mv != 0.0
        mi = m.astype(jnp.int32)
        c = plsc.cumsum(mi)
        pos = off + c - 1
        vals = lax.iota(jnp.int32, _L) + (gstart + t) * _L
        safe = m & (pos < _N_SEL)
        plsc.store_scatter(idx_v, [pos], vals, mask=safe)
        return off + jnp.sum(mi)

    lax.fori_loop(0, trip, pass_b, base, unroll=False)
    pltpu.sync_copy(idx_v, idx_sh.at[sid])
    plsc.subcore_barrier()

    _WIN = _IDX_PAD // _NW
    wbase = wid * _WIN
    pltpu.sync_copy(idx_sh.at[:, pl.ds((wid // 2) * (2 * _WIN), 2 * _WIN)],
                    win2_v)
    for o in range(2 * _WIN // _L):
        s = jnp.zeros((_L,), jnp.int32)
        for r in range(_NS):
            s = s + win2_v[r, pl.ds(o * _L, _L)]
        win_v[pl.ds(o * _L, _L)] = s
    woff = (wid % 2) * _WIN

    cap = _N_SEL - _K - wbase
    starts_l = [jnp.minimum(_K * c, cap) for c in range(_NCHUNK)]
    bufs = [buf0, buf1]
    gsems = [sem0, sem1]
    ssems = [sem2, sem3]

    def gather_start(c):
        return pltpu.async_copy(
            xt_hbm.at[win_v.at[pl.ds(woff + starts_l[c], _K)]],
            bufs[c % 2], gsems[c % 2]
        )

    def scatter_start(c):
        return pltpu.async_copy(
            bufs[c % 2], out_hbm.at[pl.ds(wbase + starts_l[c], _K), :],
            ssems[c % 2]
        )

    gcp = [None] * _NCHUNK
    scp = [None] * _NCHUNK
    gcp[0] = gather_start(0)
    for c in range(_NCHUNK):
        gcp[c].wait()
        if c + 1 < _NCHUNK:
            if c >= 1:
                scp[c - 1].wait()
            gcp[c + 1] = gather_start(c + 1)
        scp[c] = scatter_start(c)
    scp[_NCHUNK - 2].wait()
    scp[_NCHUNK - 1].wait()


@functools.partial(
    pl.kernel,
    out_type=jax.ShapeDtypeStruct((_N_SEL, _N_ROWS), jnp.float32),
    mesh=plsc.VectorSubcoreMesh(core_axis_name="c", subcore_axis_name="s"),
    compiler_params=pltpu.CompilerParams(needs_layout_passes=False),
    scratch_types=[
        pltpu.VMEM((_MASK_W,), jnp.float32),
        pltpu.VMEM((_IDX_PAD,), jnp.int32),
        pltpu.VMEM((_L,), jnp.int32),
        pltpu.VMEM((_NS, _L), jnp.int32),
        pltpu.VMEM((_NS, 2 * _IDX_PAD // _NW), jnp.int32),
        pltpu.VMEM((2 * _IDX_PAD // _NW,), jnp.int32),
        pltpu.VMEM((_K, _N_ROWS), jnp.float32),
        pltpu.VMEM((_K, _N_ROWS), jnp.float32),
        pltpu.VMEM_SHARED((_NS, _L), jnp.int32),
        pltpu.VMEM_SHARED((_NS, _IDX_PAD), jnp.int32),
        pltpu.SemaphoreType.DMA,
        pltpu.SemaphoreType.DMA,
        pltpu.SemaphoreType.DMA,
        pltpu.SemaphoreType.DMA,
    ],
)
def _gather_rows_t(xt_hbm, mask_hbm, out_hbm,
                   mask_v, idx_v, accv, cnt_tab, win2_v, win_v, buf0, buf1,
                   counts_sh, idx_sh, sem0, sem1, sem2, sem3):
    _body(xt_hbm, mask_hbm, out_hbm,
          mask_v, idx_v, accv, cnt_tab, win2_v, win_v, buf0, buf1,
          counts_sh, idx_sh, sem0, sem1, sem2, sem3)


def kernel(x, selection_mask):
    xt = jnp.transpose(x)
    out_t = _gather_rows_t(xt, selection_mask)
    return jnp.transpose(out_t)


# speedup vs baseline: 1.0458x; 1.0458x over previous
import functools

import jax
import jax.numpy as jnp
from jax import lax
from jax.experimental import pallas as pl
from jax.experimental.pallas import tpu as pltpu
from jax.experimental.pallas import tpu_sc as plsc

_N_ROWS = 4096
_N_COLS = 20000
_N_SEL = 2000
_NC = 2
_NS = 16
_L = 16
_NW = _NC * _NS
_MASK_VREGS = _N_COLS // _L
_VPW = -(-_MASK_VREGS // _NS)
_MASK_W = _VPW * _L
_IDX_PAD = 2048
_K = 8
_NCHUNK = 8


def _body(xt_hbm, mask_hbm, out_hbm,
          mask_v, idx_v, accv, cnt_tab, win2_v, win_v, buf0, buf1, buf2,
          counts_sh, idx_sh, sem0, sem1, sem2, sem3, sem4, sem5):
    cid = lax.axis_index("c")
    sid = lax.axis_index("s")
    wid = sid * _NC + cid

    dma_off = jnp.minimum(sid * _MASK_W, _N_COLS - _MASK_W)
    pltpu.sync_copy(mask_hbm.at[pl.ds(dma_off, _MASK_W)], mask_v)

    def init_body(j, carry):
        idx_v[pl.ds(j * _L, _L)] = jnp.zeros((_L,), jnp.int32)
        return carry

    lax.fori_loop(0, _IDX_PAD // _L, init_body, jnp.int32(0), unroll=False)

    gstart = sid * _VPW
    trip = jnp.minimum(_MASK_VREGS - gstart, _VPW)
    lbase_w = gstart * _L - dma_off

    def pass_a(t, acc):
        mv = mask_v[pl.ds(lbase_w + t * _L, _L)]
        return acc + (mv != 0.0).astype(jnp.int32)

    acc = lax.fori_loop(0, trip, pass_a, jnp.zeros((_L,), jnp.int32),
                        unroll=False)
    accv[...] = acc
    pltpu.sync_copy(accv, counts_sh.at[sid])
    plsc.subcore_barrier()

    pltpu.sync_copy(counts_sh, cnt_tab)
    acc2 = jnp.zeros((_L,), jnp.int32)
    for r in range(_NS):
        acc2 = acc2 + cnt_tab[r, :] * (r < sid).astype(jnp.int32)
    base = jnp.sum(acc2)

    def pass_b(t, off):
        mv = mask_v[pl.ds(lbase_w + t * _L, _L)]
        m = mv != 0.0
        mi = m.astype(jnp.int32)
        c = plsc.cumsum(mi)
        pos = off + c - 1
        vals = lax.iota(jnp.int32, _L) + (gstart + t) * _L
        safe = m & (pos < _N_SEL)
        plsc.store_scatter(idx_v, [pos], vals, mask=safe)
        return off + jnp.sum(mi)

    lax.fori_loop(0, trip, pass_b, base, unroll=False)
    pltpu.sync_copy(idx_v, idx_sh.at[sid])
    plsc.subcore_barrier()

    _WIN = _IDX_PAD // _NW
    wbase = wid * _WIN
    pltpu.sync_copy(idx_sh.at[:, pl.ds((wid // 2) * (2 * _WIN), 2 * _WIN)],
                    win2_v)
    for o in range(2 * _WIN // _L):
        s = jnp.zeros((_L,), jnp.int32)
        for r in range(_NS):
            s = s + win2_v[r, pl.ds(o * _L, _L)]
        win_v[pl.ds(o * _L, _L)] = s
    woff = (wid % 2) * _WIN

    cap = _N_SEL - _K - wbase
    starts_l = [jnp.minimum(_K * c, cap) for c in range(_NCHUNK)]
    bufs = [buf0, buf1, buf2]
    gsems = [sem0, sem1, sem2]
    ssems = [sem3, sem4, sem5]

    def gather_start(c):
        return pltpu.async_copy(
            xt_hbm.at[win_v.at[pl.ds(woff + starts_l[c], _K)]],
            bufs[c % 3], gsems[c % 3]
        )

    def scatter_start(c):
        return pltpu.async_copy(
            bufs[c % 3], out_hbm.at[pl.ds(wbase + starts_l[c], _K), :],
            ssems[c % 3]
        )

    gcp = [None] * _NCHUNK
    scp = [None] * _NCHUNK
    gcp[0] = gather_start(0)
    gcp[1] = gather_start(1)
    for c in range(_NCHUNK):
        gcp[c].wait()
        if c + 2 < _NCHUNK:
            if c >= 1:
                scp[c - 1].wait()
            gcp[c + 2] = gather_start(c + 2)
        scp[c] = scatter_start(c)
    for c in range(_NCHUNK - 3, _NCHUNK):
        scp[c].wait()


@functools.partial(
    pl.kernel,
    out_type=jax.ShapeDtypeStruct((_N_SEL, _N_ROWS), jnp.float32),
    mesh=plsc.VectorSubcoreMesh(core_axis_name="c", subcore_axis_name="s"),
    compiler_params=pltpu.CompilerParams(needs_layout_passes=False),
    scratch_types=[
        pltpu.VMEM((_MASK_W,), jnp.float32),
        pltpu.VMEM((_IDX_PAD,), jnp.int32),
        pltpu.VMEM((_L,), jnp.int32),
        pltpu.VMEM((_NS, _L), jnp.int32),
        pltpu.VMEM((_NS, 2 * _IDX_PAD // _NW), jnp.int32),
        pltpu.VMEM((2 * _IDX_PAD // _NW,), jnp.int32),
        pltpu.VMEM((_K, _N_ROWS), jnp.float32),
        pltpu.VMEM((_K, _N_ROWS), jnp.float32),
        pltpu.VMEM((_K, _N_ROWS), jnp.float32),
        pltpu.VMEM_SHARED((_NS, _L), jnp.int32),
        pltpu.VMEM_SHARED((_NS, _IDX_PAD), jnp.int32),
        pltpu.SemaphoreType.DMA,
        pltpu.SemaphoreType.DMA,
        pltpu.SemaphoreType.DMA,
        pltpu.SemaphoreType.DMA,
        pltpu.SemaphoreType.DMA,
        pltpu.SemaphoreType.DMA,
    ],
)
def _gather_rows_t(xt_hbm, mask_hbm, out_hbm,
                   mask_v, idx_v, accv, cnt_tab, win2_v, win_v, buf0, buf1, buf2,
                   counts_sh, idx_sh, sem0, sem1, sem2, sem3, sem4, sem5):
    _body(xt_hbm, mask_hbm, out_hbm,
          mask_v, idx_v, accv, cnt_tab, win2_v, win_v, buf0, buf1, buf2,
          counts_sh, idx_sh, sem0, sem1, sem2, sem3, sem4, sem5)


def kernel(x, selection_mask):
    xt = jnp.transpose(x)
    out_t = _gather_rows_t(xt, selection_mask)
    return jnp.transpose(out_t)


# speedup vs baseline: 1.0479x; 1.0019x over previous
import functools

import jax
import jax.numpy as jnp
from jax import lax
from jax.experimental import pallas as pl
from jax.experimental.pallas import tpu as pltpu
from jax.experimental.pallas import tpu_sc as plsc

_N_ROWS = 4096
_N_COLS = 20000
_N_SEL = 2000
_NC = 2
_NS = 16
_L = 16
_NW = _NC * _NS
_MASK_VREGS = _N_COLS // _L
_VPW = -(-_MASK_VREGS // _NS)
_MASK_W = _VPW * _L
_IDX_PAD = 2048
_K = 8
_NCHUNK = 8


def _body(xt_hbm, mask_hbm, out_hbm,
          mask_v, idx_v, accv, cnt_tab, win2_v, win_v, buf0, buf1, buf2,
          counts_sh, idx_sh, sem0, sem1, sem2, sem3, sem4, sem5):
    cid = lax.axis_index("c")
    sid = lax.axis_index("s")
    wid = sid * _NC + cid

    dma_off = jnp.minimum(sid * _MASK_W, _N_COLS - _MASK_W)
    pltpu.sync_copy(mask_hbm.at[pl.ds(dma_off, _MASK_W)], mask_v)

    def init_body(j, carry):
        idx_v[pl.ds(j * _L, _L)] = jnp.zeros((_L,), jnp.int32)
        return carry

    lax.fori_loop(0, _IDX_PAD // _L, init_body, jnp.int32(0), unroll=False)

    gstart = sid * _VPW
    trip = jnp.minimum(_MASK_VREGS - gstart, _VPW)
    lbase_w = gstart * _L - dma_off

    def pass_a(t, acc):
        mv = mask_v[pl.ds(lbase_w + t * _L, _L)]
        return acc + (mv != 0.0).astype(jnp.int32)

    acc = lax.fori_loop(0, trip, pass_a, jnp.zeros((_L,), jnp.int32),
                        unroll=False)
    accv[...] = acc
    pltpu.sync_copy(accv, counts_sh.at[sid])
    plsc.subcore_barrier()

    pltpu.sync_copy(counts_sh, cnt_tab)
    acc2 = jnp.zeros((_L,), jnp.int32)
    for r in range(_NS):
        acc2 = acc2 + cnt_tab[r, :] * (r < sid).astype(jnp.int32)
    base = jnp.sum(acc2)

    def pass_b(t, off):
        mv = mask_v[pl.ds(lbase_w + t * _L, _L)]
        m = mv != 0.0
        mi = m.astype(jnp.int32)
        c = plsc.cumsum(mi)
        pos = off + c - 1
        vals = lax.iota(jnp.int32, _L) + (gstart + t) * _L
        safe = m & (pos < _N_SEL)
        plsc.store_scatter(idx_v, [pos], vals, mask=safe)
        return off + jnp.sum(mi)

    lax.fori_loop(0, trip, pass_b, base, unroll=False)
    pltpu.sync_copy(idx_v, idx_sh.at[sid])
    plsc.subcore_barrier()

    _WIN = _IDX_PAD // _NW
    wbase = wid * _WIN
    pltpu.sync_copy(idx_sh.at[:, pl.ds((wid // 2) * (2 * _WIN), 2 * _WIN)],
                    win2_v)
    for o in range(2 * _WIN // _L):
        s = jnp.zeros((_L,), jnp.int32)
        for r in range(_NS):
            s = s + win2_v[r, pl.ds(o * _L, _L)]
        win_v[pl.ds(o * _L, _L)] = s
    woff = (wid % 2) * _WIN

    cap = _N_SEL - _K - wbase
    starts_l = [jnp.minimum(_K * c, cap) for c in range(_NCHUNK)]
    bufs = [buf0, buf1, buf2]
    gsems = [sem0, sem1, sem2]
    ssems = [sem3, sem4, sem5]

    def gather_start(c):
        return pltpu.async_copy(
            xt_hbm.at[win_v.at[pl.ds(woff + starts_l[c], _K)]],
            bufs[c % 3], gsems[c % 3]
        )

    def scatter_start(c):
        return pltpu.async_copy(
            bufs[c % 3], out_hbm.at[pl.ds(wbase + starts_l[c], _K), :],
            ssems[c % 3]
        )

    gcp = [None] * _NCHUNK
    scp = [None] * _NCHUNK
    gcp[0] = gather_start(0)
    gcp[1] = gather_start(1)
    for c in range(_NCHUNK):
        gcp[c].wait()
        if c + 2 < _NCHUNK:
            if c >= 1:
                scp[c - 1].wait()
            gcp[c + 2] = gather_start(c + 2)
        scp[c] = scatter_start(c)
    for c in range(_NCHUNK - 3, _NCHUNK):
        scp[c].wait()


@functools.partial(
    pl.kernel,
    out_type=jax.ShapeDtypeStruct((_N_SEL, _N_ROWS), jnp.float32),
    mesh=plsc.VectorSubcoreMesh(core_axis_name="c", subcore_axis_name="s"),
    compiler_params=pltpu.CompilerParams(needs_layout_passes=False,
                                         skip_device_barrier=True),
    scratch_types=[
        pltpu.VMEM((_MASK_W,), jnp.float32),
        pltpu.VMEM((_IDX_PAD,), jnp.int32),
        pltpu.VMEM((_L,), jnp.int32),
        pltpu.VMEM((_NS, _L), jnp.int32),
        pltpu.VMEM((_NS, 2 * _IDX_PAD // _NW), jnp.int32),
        pltpu.VMEM((2 * _IDX_PAD // _NW,), jnp.int32),
        pltpu.VMEM((_K, _N_ROWS), jnp.float32),
        pltpu.VMEM((_K, _N_ROWS), jnp.float32),
        pltpu.VMEM((_K, _N_ROWS), jnp.float32),
        pltpu.VMEM_SHARED((_NS, _L), jnp.int32),
        pltpu.VMEM_SHARED((_NS, _IDX_PAD), jnp.int32),
        pltpu.SemaphoreType.DMA,
        pltpu.SemaphoreType.DMA,
        pltpu.SemaphoreType.DMA,
        pltpu.SemaphoreType.DMA,
        pltpu.SemaphoreType.DMA,
        pltpu.SemaphoreType.DMA,
    ],
)
def _gather_rows_t(xt_hbm, mask_hbm, out_hbm,
                   mask_v, idx_v, accv, cnt_tab, win2_v, win_v, buf0, buf1, buf2,
                   counts_sh, idx_sh, sem0, sem1, sem2, sem3, sem4, sem5):
    _body(xt_hbm, mask_hbm, out_hbm,
          mask_v, idx_v, accv, cnt_tab, win2_v, win_v, buf0, buf1, buf2,
          counts_sh, idx_sh, sem0, sem1, sem2, sem3, sem4, sem5)


def kernel(x, selection_mask):
    xt = jnp.transpose(x)
    out_t = _gather_rows_t(xt, selection_mask)
    return jnp.transpose(out_t)
